# Initial kernel scaffold; baseline (speedup 1.0000x reference)
import jax
import jax.numpy as jnp
from jax.experimental import pallas as pl


def kernel(x, W):
    raise NotImplementedError("write your pallas kernel here")


# speedup vs baseline: 1.3996x; 1.3996x over previous
import functools
import math

import jax
import jax.numpy as jnp
from jax import lax
from jax.experimental import pallas as pl
from jax.experimental.pallas import tpu as pltpu
from jax.experimental.pallas import tpu_sc as plsc

BATCH = 4096
HIST = 200
EMB_DIM = 32
TOTAL = BATCH * HIST
SCALE = math.sqrt(EMB_DIM)

_info = plsc.get_sparse_core_info()
NC = _info.num_cores
NS = _info.num_subcores
LANES = _info.num_lanes
NW = NC * NS
PER_W = TOTAL // NW
CHUNK = 1024
N_CHUNKS = PER_W // CHUNK

_mesh = plsc.VectorSubcoreMesh(core_axis_name="c", subcore_axis_name="s")


@functools.partial(
    pl.kernel,
    mesh=_mesh,
    compiler_params=pltpu.CompilerParams(use_tc_tiling_on_sc=False),
    out_type=jax.ShapeDtypeStruct((TOTAL, EMB_DIM), jnp.float32),
    scratch_types=[
        pltpu.VMEM((CHUNK,), jnp.int32),
        pltpu.VMEM((CHUNK, EMB_DIM), jnp.float32),
        pltpu.SemaphoreType.DMA,
    ],
)
def _embed(w_hbm, x_hbm, out_hbm, idx_v, rows_v, sem):
    wid = lax.axis_index("s") * NC + lax.axis_index("c")
    base = wid * PER_W

    def chunk_body(g, carry):
        off = base + g * CHUNK
        pltpu.sync_copy(x_hbm.at[pl.ds(off, CHUNK)], idx_v)
        pltpu.async_copy(w_hbm.at[idx_v], rows_v, sem).wait()

        def row_body(i, c):
            rows_v[i, pl.ds(0, LANES)] = rows_v[i, pl.ds(0, LANES)] * SCALE
            rows_v[i, pl.ds(LANES, LANES)] = (
                rows_v[i, pl.ds(LANES, LANES)] * SCALE
            )
            return c

        lax.fori_loop(0, CHUNK, row_body, 0, unroll=4)
        pltpu.sync_copy(rows_v, out_hbm.at[pl.ds(off, CHUNK)])
        return carry

    lax.fori_loop(0, N_CHUNKS, chunk_body, 0)


def kernel(x, W):
    out = _embed(W, x.reshape(TOTAL))
    return out.reshape(BATCH, HIST, EMB_DIM)


# speedup vs baseline: 1.4802x; 1.0576x over previous
import functools
import math

import jax
import jax.numpy as jnp
from jax import lax
from jax.experimental import pallas as pl
from jax.experimental.pallas import tpu as pltpu
from jax.experimental.pallas import tpu_sc as plsc

BATCH = 4096
HIST = 200
EMB_DIM = 32
TOTAL = BATCH * HIST
SCALE = math.sqrt(EMB_DIM)

_info = plsc.get_sparse_core_info()
NC = _info.num_cores
NS = _info.num_subcores
LANES = _info.num_lanes
NW = NC * NS
PER_W = TOTAL // NW
CHUNK = 640
N_CHUNKS = PER_W // CHUNK
NBUF = 4
LOOKAHEAD = 2

_mesh = plsc.VectorSubcoreMesh(core_axis_name="c", subcore_axis_name="s")


@functools.partial(
    pl.kernel,
    mesh=_mesh,
    compiler_params=pltpu.CompilerParams(use_tc_tiling_on_sc=False),
    out_type=jax.ShapeDtypeStruct((TOTAL, EMB_DIM), jnp.float32),
    scratch_types=[
        pltpu.VMEM((PER_W,), jnp.int32),
        [pltpu.VMEM((CHUNK, EMB_DIM), jnp.float32) for _ in range(NBUF)],
        [pltpu.SemaphoreType.DMA for _ in range(NBUF)],
        [pltpu.SemaphoreType.DMA for _ in range(NBUF)],
    ],
)
def _embed(w_hbm, x_hbm, out_hbm, idx_v, rows, gsems, ssems):
    wid = lax.axis_index("s") * NC + lax.axis_index("c")
    base = wid * PER_W

    pltpu.sync_copy(x_hbm.at[pl.ds(base, PER_W)], idx_v)

    def start_gather(c, b):
        pltpu.async_copy(
            w_hbm.at[idx_v.at[pl.ds(c * CHUNK, CHUNK)]], rows[b], gsems[b]
        )

    def wait_gather(b):
        pltpu.make_async_copy(
            w_hbm.at[idx_v.at[pl.ds(0, CHUNK)]], rows[b], gsems[b]
        ).wait()

    def start_store(c, b):
        pltpu.async_copy(
            rows[b], out_hbm.at[pl.ds(base + c * CHUNK, CHUNK)], ssems[b]
        )

    def wait_store(b):
        pltpu.make_async_copy(
            rows[b], out_hbm.at[pl.ds(base, CHUNK)], ssems[b]
        ).wait()

    for c in range(LOOKAHEAD):
        start_gather(c, c % NBUF)

    def loop_body(t, carry):
        for b in range(NBUF):
            c = t * NBUF + b
            c_next = c + LOOKAHEAD
            nb = (b + LOOKAHEAD) % NBUF

            @pl.when(jnp.logical_and(c_next >= NBUF, c_next < N_CHUNKS))
            def _():
                wait_store(nb)

            @pl.when(c_next < N_CHUNKS)
            def _():
                start_gather(c_next, nb)

            wait_gather(b)

            @plsc.parallel_loop(0, CHUNK, unroll=8)
            def _(i):
                rows[b][i, pl.ds(0, LANES)] = (
                    rows[b][i, pl.ds(0, LANES)] * SCALE
                )
                rows[b][i, pl.ds(LANES, LANES)] = (
                    rows[b][i, pl.ds(LANES, LANES)] * SCALE
                )

            start_store(c, b)
        return carry

    lax.fori_loop(0, N_CHUNKS // NBUF, loop_body, 0)

    for b in range(NBUF):
        wait_store(b)


def kernel(x, W):
    out = _embed(W, x.reshape(TOTAL))
    return out.reshape(BATCH, HIST, EMB_DIM)


# speedup vs baseline: 1.5116x; 1.0212x over previous
import functools
import math

import jax
import jax.numpy as jnp
from jax import lax
from jax.experimental import pallas as pl
from jax.experimental.pallas import tpu as pltpu
from jax.experimental.pallas import tpu_sc as plsc

BATCH = 4096
HIST = 200
EMB_DIM = 32
VOC = 1000000
TOTAL = BATCH * HIST
SCALE = math.sqrt(EMB_DIM)

_info = plsc.get_sparse_core_info()
NC = _info.num_cores
NS = _info.num_subcores
LANES = _info.num_lanes
NW = NC * NS

_mesh = plsc.VectorSubcoreMesh(core_axis_name="c", subcore_axis_name="s")


RBLK = 128
N_FULL = (VOC // RBLK)
UNITS_EACH = N_FULL // NW
UNITS_REM = N_FULL % NW
TAIL_R0 = N_FULL * RBLK
TAIL_N = VOC - TAIL_R0


@functools.partial(
    pl.kernel,
    mesh=_mesh,
    compiler_params=pltpu.CompilerParams(
        use_tc_tiling_on_sc=True, needs_layout_passes=False
    ),
    out_type=jax.ShapeDtypeStruct((VOC * EMB_DIM,), jnp.float32),
    scratch_types=[
        [pltpu.VMEM((EMB_DIM, RBLK), jnp.float32) for _ in range(2)],
        [pltpu.VMEM((RBLK * EMB_DIM,), jnp.float32) for _ in range(2)],
        pltpu.VMEM((TAIL_N * EMB_DIM,), jnp.float32),
        [pltpu.SemaphoreType.DMA for _ in range(2)],
        [pltpu.SemaphoreType.DMA for _ in range(2)],
        pltpu.SemaphoreType.DMA,
    ],
)
def _untile(wt_hbm, wtail_hbm, wrow_hbm, tv, ov, tov, gsems, wsems, tsem):
    wid = lax.axis_index("s") * NC + lax.axis_index("c")
    iota = lax.iota(jnp.int32, LANES)
    iota_hi = iota + LANES

    def col_of(k):
        return k * NW + wid

    def start_read(k, p):
        tc = col_of(k)
        pltpu.async_copy(
            wt_hbm.at[:, pl.ds(tc * RBLK, RBLK)], tv[p], gsems[p]
        )

    def wait_read(p):
        pltpu.make_async_copy(
            wt_hbm.at[:, pl.ds(0, RBLK)], tv[p], gsems[p]
        ).wait()

    def untile_block(p):
        @plsc.parallel_loop(0, RBLK, unroll=8)
        def _(rs):
            cs = jnp.full((LANES,), rs, jnp.int32)
            lo = plsc.load_gather(tv[p], [iota, cs])
            hi = plsc.load_gather(tv[p], [iota_hi, cs])
            ov[p][pl.ds(rs * EMB_DIM, LANES)] = lo
            ov[p][pl.ds(rs * EMB_DIM + LANES, LANES)] = hi

    def start_write(k, p):
        tc = col_of(k)
        pltpu.async_copy(
            ov[p], wrow_hbm.at[pl.ds(tc * (RBLK * EMB_DIM), RBLK * EMB_DIM)],
            wsems[p],
        )

    def wait_write(p):
        pltpu.make_async_copy(
            ov[p], wrow_hbm.at[pl.ds(0, RBLK * EMB_DIM)], wsems[p]
        ).wait()

    n_units = UNITS_EACH + 1
    start_read(0, 0)

    def body(k, carry):
        for p in range(2):
            kk = k * 2 + p

            @pl.when(kk < UNITS_EACH)
            def _():
                @pl.when(kk + 1 < n_units)
                def _():
                    do_next = jnp.logical_or(
                        kk + 1 < UNITS_EACH, wid < UNITS_REM
                    )

                    @pl.when(do_next)
                    def _():
                        start_read(kk + 1, 1 - p)

                wait_read(p)

                @pl.when(kk >= 2)
                def _():
                    wait_write(p)

                untile_block(p)
                start_write(kk, p)
        return carry

    lax.fori_loop(0, (UNITS_EACH + 1) // 2, body, 0)

    p_x = UNITS_EACH % 2

    @pl.when(wid < UNITS_REM)
    def _():
        wait_read(p_x)
        wait_write(p_x)
        untile_block(p_x)
        start_write(UNITS_EACH, p_x)

    @pl.when(wid == NW - 1)
    def _():
        pltpu.sync_copy(wtail_hbm, tov)
        pltpu.async_copy(
            tov, wrow_hbm.at[pl.ds(TAIL_R0 * EMB_DIM, TAIL_N * EMB_DIM)],
            tsem,
        )
        pltpu.make_async_copy(
            tov, wrow_hbm.at[pl.ds(0, TAIL_N * EMB_DIM)], tsem
        ).wait()

    for p in range(2):
        wait_write(p)


PER_W = TOTAL // NW
CHUNK = 640
N_CHUNKS = PER_W // CHUNK
NBUF = 4
LOOKAHEAD = 2


@functools.partial(
    pl.kernel,
    mesh=_mesh,
    compiler_params=pltpu.CompilerParams(use_tc_tiling_on_sc=False),
    out_type=jax.ShapeDtypeStruct((TOTAL, EMB_DIM), jnp.float32),
    scratch_types=[
        pltpu.VMEM((PER_W,), jnp.int32),
        [pltpu.VMEM((CHUNK, EMB_DIM), jnp.float32) for _ in range(NBUF)],
        [pltpu.SemaphoreType.DMA for _ in range(NBUF)],
        [pltpu.SemaphoreType.DMA for _ in range(NBUF)],
    ],
)
def _embed(w_hbm, x_hbm, out_hbm, idx_v, rows, gsems, ssems):
    wid = lax.axis_index("s") * NC + lax.axis_index("c")
    base = wid * PER_W

    pltpu.sync_copy(x_hbm.at[pl.ds(base, PER_W)], idx_v)

    def start_gather(c, b):
        pltpu.async_copy(
            w_hbm.at[idx_v.at[pl.ds(c * CHUNK, CHUNK)]], rows[b], gsems[b]
        )

    def wait_gather(b):
        pltpu.make_async_copy(
            w_hbm.at[idx_v.at[pl.ds(0, CHUNK)]], rows[b], gsems[b]
        ).wait()

    def start_store(c, b):
        pltpu.async_copy(
            rows[b], out_hbm.at[pl.ds(base + c * CHUNK, CHUNK)], ssems[b]
        )

    def wait_store(b):
        pltpu.make_async_copy(
            rows[b], out_hbm.at[pl.ds(base, CHUNK)], ssems[b]
        ).wait()

    for c in range(LOOKAHEAD):
        start_gather(c, c % NBUF)

    def loop_body(t, carry):
        for b in range(NBUF):
            c = t * NBUF + b
            c_next = c + LOOKAHEAD
            nb = (b + LOOKAHEAD) % NBUF

            @pl.when(jnp.logical_and(c_next >= NBUF, c_next < N_CHUNKS))
            def _():
                wait_store(nb)

            @pl.when(c_next < N_CHUNKS)
            def _():
                start_gather(c_next, nb)

            wait_gather(b)

            @plsc.parallel_loop(0, CHUNK, unroll=8)
            def _(i):
                rows[b][i, pl.ds(0, LANES)] = (
                    rows[b][i, pl.ds(0, LANES)] * SCALE
                )
                rows[b][i, pl.ds(LANES, LANES)] = (
                    rows[b][i, pl.ds(LANES, LANES)] * SCALE
                )

            start_store(c, b)
        return carry

    lax.fori_loop(0, N_CHUNKS // NBUF, loop_body, 0)

    for b in range(NBUF):
        wait_store(b)


def kernel(x, W):
    wrow = _untile(jnp.transpose(W), W[TAIL_R0:].reshape(TAIL_N * EMB_DIM))
    out = _embed(wrow.reshape(VOC, EMB_DIM), x.reshape(TOTAL))
    return out.reshape(BATCH, HIST, EMB_DIM)


# speedup vs baseline: 1.5288x; 1.0114x over previous
import functools
import math

import jax
import jax.numpy as jnp
from jax import lax
from jax.experimental import pallas as pl
from jax.experimental.pallas import tpu as pltpu
from jax.experimental.pallas import tpu_sc as plsc

BATCH = 4096
HIST = 200
EMB_DIM = 32
VOC = 1000000
TOTAL = BATCH * HIST
SCALE = math.sqrt(EMB_DIM)

_info = plsc.get_sparse_core_info()
NC = _info.num_cores
NS = _info.num_subcores
LANES = _info.num_lanes
NW = NC * NS

_mesh = plsc.VectorSubcoreMesh(core_axis_name="c", subcore_axis_name="s")


RBLK = 128
N_FULL = (VOC // RBLK)
UNITS_EACH = N_FULL // NW
UNITS_REM = N_FULL % NW
TAIL_R0 = N_FULL * RBLK
TAIL_N = VOC - TAIL_R0


@functools.partial(
    pl.kernel,
    mesh=_mesh,
    compiler_params=pltpu.CompilerParams(
        use_tc_tiling_on_sc=True, needs_layout_passes=False
    ),
    out_type=jax.ShapeDtypeStruct((VOC * EMB_DIM,), jnp.float32),
    scratch_types=[
        [pltpu.VMEM((EMB_DIM, RBLK), jnp.float32) for _ in range(2)],
        [pltpu.VMEM((RBLK * EMB_DIM,), jnp.float32) for _ in range(2)],
        pltpu.VMEM((TAIL_N * EMB_DIM,), jnp.float32),
        [pltpu.SemaphoreType.DMA for _ in range(2)],
        [pltpu.SemaphoreType.DMA for _ in range(2)],
        pltpu.SemaphoreType.DMA,
    ],
)
def _untile(wt_hbm, wtail_hbm, wrow_hbm, tv, ov, tov, gsems, wsems, tsem):
    wid = lax.axis_index("s") * NC + lax.axis_index("c")
    iota = lax.iota(jnp.int32, LANES)
    iota_hi = iota + LANES

    def col_of(k):
        return k * NW + wid

    def start_read(k, p):
        tc = col_of(k)
        pltpu.async_copy(
            wt_hbm.at[:, pl.ds(tc * RBLK, RBLK)], tv[p], gsems[p]
        )

    def wait_read(p):
        pltpu.make_async_copy(
            wt_hbm.at[:, pl.ds(0, RBLK)], tv[p], gsems[p]
        ).wait()

    iota32 = iota * EMB_DIM

    def untile_block(p):
        @plsc.parallel_loop(0, (RBLK // LANES) * EMB_DIM, unroll=8)
        def _(j):
            d = j & (EMB_DIM - 1)
            blk = j >> 5
            vals = tv[p][d, pl.ds(blk * LANES, LANES)]
            plsc.store_scatter(
                ov[p], [iota32 + (blk * (LANES * EMB_DIM) + d)], vals
            )

    def start_write(k, p):
        tc = col_of(k)
        pltpu.async_copy(
            ov[p], wrow_hbm.at[pl.ds(tc * (RBLK * EMB_DIM), RBLK * EMB_DIM)],
            wsems[p],
        )

    def wait_write(p):
        pltpu.make_async_copy(
            ov[p], wrow_hbm.at[pl.ds(0, RBLK * EMB_DIM)], wsems[p]
        ).wait()

    n_units = UNITS_EACH + 1
    start_read(0, 0)

    def body(k, carry):
        for p in range(2):
            kk = k * 2 + p

            @pl.when(kk < UNITS_EACH)
            def _():
                @pl.when(kk + 1 < n_units)
                def _():
                    do_next = jnp.logical_or(
                        kk + 1 < UNITS_EACH, wid < UNITS_REM
                    )

                    @pl.when(do_next)
                    def _():
                        start_read(kk + 1, 1 - p)

                wait_read(p)

                @pl.when(kk >= 2)
                def _():
                    wait_write(p)

                untile_block(p)
                start_write(kk, p)
        return carry

    lax.fori_loop(0, (UNITS_EACH + 1) // 2, body, 0)

    p_x = UNITS_EACH % 2

    @pl.when(wid < UNITS_REM)
    def _():
        wait_read(p_x)
        wait_write(p_x)
        untile_block(p_x)
        start_write(UNITS_EACH, p_x)

    @pl.when(wid == NW - 1)
    def _():
        pltpu.sync_copy(wtail_hbm, tov)
        pltpu.async_copy(
            tov, wrow_hbm.at[pl.ds(TAIL_R0 * EMB_DIM, TAIL_N * EMB_DIM)],
            tsem,
        )
        pltpu.make_async_copy(
            tov, wrow_hbm.at[pl.ds(0, TAIL_N * EMB_DIM)], tsem
        ).wait()

    for p in range(2):
        wait_write(p)


PER_W = TOTAL // NW
CHUNK = 640
N_CHUNKS = PER_W // CHUNK
NBUF = 4
LOOKAHEAD = 2


@functools.partial(
    pl.kernel,
    mesh=_mesh,
    compiler_params=pltpu.CompilerParams(use_tc_tiling_on_sc=False),
    out_type=jax.ShapeDtypeStruct((TOTAL, EMB_DIM), jnp.float32),
    scratch_types=[
        pltpu.VMEM((PER_W,), jnp.int32),
        [pltpu.VMEM((CHUNK, EMB_DIM), jnp.float32) for _ in range(NBUF)],
        [pltpu.SemaphoreType.DMA for _ in range(NBUF)],
        [pltpu.SemaphoreType.DMA for _ in range(NBUF)],
    ],
)
def _embed(w_hbm, x_hbm, out_hbm, idx_v, rows, gsems, ssems):
    wid = lax.axis_index("s") * NC + lax.axis_index("c")
    base = wid * PER_W

    pltpu.sync_copy(x_hbm.at[pl.ds(base, PER_W)], idx_v)

    def start_gather(c, b):
        pltpu.async_copy(
            w_hbm.at[idx_v.at[pl.ds(c * CHUNK, CHUNK)]], rows[b], gsems[b]
        )

    def wait_gather(b):
        pltpu.make_async_copy(
            w_hbm.at[idx_v.at[pl.ds(0, CHUNK)]], rows[b], gsems[b]
        ).wait()

    def start_store(c, b):
        pltpu.async_copy(
            rows[b], out_hbm.at[pl.ds(base + c * CHUNK, CHUNK)], ssems[b]
        )

    def wait_store(b):
        pltpu.make_async_copy(
            rows[b], out_hbm.at[pl.ds(base, CHUNK)], ssems[b]
        ).wait()

    for c in range(LOOKAHEAD):
        start_gather(c, c % NBUF)

    def loop_body(t, carry):
        for b in range(NBUF):
            c = t * NBUF + b
            c_next = c + LOOKAHEAD
            nb = (b + LOOKAHEAD) % NBUF

            @pl.when(jnp.logical_and(c_next >= NBUF, c_next < N_CHUNKS))
            def _():
                wait_store(nb)

            @pl.when(c_next < N_CHUNKS)
            def _():
                start_gather(c_next, nb)

            wait_gather(b)

            @plsc.parallel_loop(0, CHUNK, unroll=8)
            def _(i):
                rows[b][i, pl.ds(0, LANES)] = (
                    rows[b][i, pl.ds(0, LANES)] * SCALE
                )
                rows[b][i, pl.ds(LANES, LANES)] = (
                    rows[b][i, pl.ds(LANES, LANES)] * SCALE
                )

            start_store(c, b)
        return carry

    lax.fori_loop(0, N_CHUNKS // NBUF, loop_body, 0)

    for b in range(NBUF):
        wait_store(b)


def kernel(x, W):
    wrow = _untile(jnp.transpose(W), W[TAIL_R0:].reshape(TAIL_N * EMB_DIM))
    out = _embed(wrow.reshape(VOC, EMB_DIM), x.reshape(TOTAL))
    return out.reshape(BATCH, HIST, EMB_DIM)


# speedup vs baseline: 1.6585x; 1.0848x over previous
import functools
import math

import jax
import jax.numpy as jnp
from jax import lax
from jax.experimental import pallas as pl
from jax.experimental.pallas import tpu as pltpu
from jax.experimental.pallas import tpu_sc as plsc

BATCH = 4096
HIST = 200
EMB_DIM = 32
VOC = 1000000
TOTAL = BATCH * HIST
SCALE = math.sqrt(EMB_DIM)

_info = plsc.get_sparse_core_info()
NC = _info.num_cores
NS = _info.num_subcores
LANES = _info.num_lanes
NW = NC * NS

_mesh = plsc.VectorSubcoreMesh(core_axis_name="c", subcore_axis_name="s")


RBLK = 128
N_FULL = (VOC // RBLK)
UNITS_EACH = N_FULL // NW
UNITS_REM = N_FULL % NW
TAIL_R0 = N_FULL * RBLK
TAIL_N = VOC - TAIL_R0


@functools.partial(
    pl.kernel,
    mesh=_mesh,
    compiler_params=pltpu.CompilerParams(
        use_tc_tiling_on_sc=True, needs_layout_passes=False
    ),
    out_type=jax.ShapeDtypeStruct((VOC * EMB_DIM,), jnp.float32),
    scratch_types=[
        [pltpu.VMEM((EMB_DIM, RBLK), jnp.float32) for _ in range(2)],
        [pltpu.VMEM((RBLK * EMB_DIM,), jnp.float32) for _ in range(2)],
        pltpu.VMEM((TAIL_N * EMB_DIM,), jnp.float32),
        [pltpu.SemaphoreType.DMA for _ in range(2)],
        [pltpu.SemaphoreType.DMA for _ in range(2)],
        pltpu.SemaphoreType.DMA,
    ],
)
def _untile(wt_hbm, wtail_hbm, wrow_hbm, tv, ov, tov, gsems, wsems, tsem):
    wid = lax.axis_index("s") * NC + lax.axis_index("c")
    iota = lax.iota(jnp.int32, LANES)
    iota_hi = iota + LANES

    def col_of(k):
        return k * NW + wid

    def start_read(k, p):
        tc = col_of(k)
        pltpu.async_copy(
            wt_hbm.at[:, pl.ds(tc * RBLK, RBLK)], tv[p], gsems[p]
        )

    def wait_read(p):
        pltpu.make_async_copy(
            wt_hbm.at[:, pl.ds(0, RBLK)], tv[p], gsems[p]
        ).wait()

    iota32 = iota * EMB_DIM

    def untile_block(p):
        @plsc.parallel_loop(0, (RBLK // LANES) * EMB_DIM, unroll=8)
        def _(j):
            d = j & (EMB_DIM - 1)
            blk = j >> 5
            vals = tv[p][d, pl.ds(blk * LANES, LANES)]
            plsc.store_scatter(
                ov[p], [iota32 + (blk * (LANES * EMB_DIM) + d)], vals
            )

    def start_write(k, p):
        tc = col_of(k)
        pltpu.async_copy(
            ov[p], wrow_hbm.at[pl.ds(tc * (RBLK * EMB_DIM), RBLK * EMB_DIM)],
            wsems[p],
        )

    def wait_write(p):
        pltpu.make_async_copy(
            ov[p], wrow_hbm.at[pl.ds(0, RBLK * EMB_DIM)], wsems[p]
        ).wait()

    n_units = UNITS_EACH + 1
    start_read(0, 0)

    def body(k, carry):
        for p in range(2):
            kk = k * 2 + p

            @pl.when(kk < UNITS_EACH)
            def _():
                @pl.when(kk + 1 < n_units)
                def _():
                    do_next = jnp.logical_or(
                        kk + 1 < UNITS_EACH, wid < UNITS_REM
                    )

                    @pl.when(do_next)
                    def _():
                        start_read(kk + 1, 1 - p)

                wait_read(p)

                @pl.when(kk >= 2)
                def _():
                    wait_write(p)

                untile_block(p)
                start_write(kk, p)
        return carry

    lax.fori_loop(0, (UNITS_EACH + 1) // 2, body, 0)

    p_x = UNITS_EACH % 2

    @pl.when(wid < UNITS_REM)
    def _():
        wait_read(p_x)
        wait_write(p_x)
        untile_block(p_x)
        start_write(UNITS_EACH, p_x)

    @pl.when(wid == NW - 1)
    def _():
        pltpu.sync_copy(wtail_hbm, tov)
        pltpu.async_copy(
            tov, wrow_hbm.at[pl.ds(TAIL_R0 * EMB_DIM, TAIL_N * EMB_DIM)],
            tsem,
        )
        pltpu.make_async_copy(
            tov, wrow_hbm.at[pl.ds(0, TAIL_N * EMB_DIM)], tsem
        ).wait()

    for p in range(2):
        wait_write(p)


PER_W = TOTAL // NW
HBLK = 128
N_TR = EMB_DIM // 8
UNITS_B = PER_W // HBLK
N_TC = BATCH // HBLK


@functools.partial(
    pl.kernel,
    mesh=_mesh,
    compiler_params=pltpu.CompilerParams(
        use_tc_tiling_on_sc=False, needs_layout_passes=False
    ),
    out_type=jax.ShapeDtypeStruct((HIST, N_TR, N_TC, 8 * HBLK), jnp.float32),
    scratch_types=[
        pltpu.VMEM((PER_W,), jnp.int32),
        [pltpu.VMEM((HBLK, EMB_DIM), jnp.float32) for _ in range(2)],
        [pltpu.VMEM((EMB_DIM * HBLK,), jnp.float32) for _ in range(2)],
        [pltpu.SemaphoreType.DMA for _ in range(2)],
        [pltpu.SemaphoreType.DMA for _ in range(2)],
    ],
)
def _embed(w_hbm, x_hbm, out_hbm, idx_v, rbuf, tbuf, gsems, ssems):
    wid = lax.axis_index("s") * NC + lax.axis_index("c")
    base = wid * PER_W
    iota = lax.iota(jnp.int32, LANES)
    iota128 = iota * HBLK

    pltpu.sync_copy(x_hbm.at[pl.ds(base, PER_W)], idx_v)

    def start_gather(k, p):
        pltpu.async_copy(
            w_hbm.at[idx_v.at[pl.ds(k * HBLK, HBLK)]], rbuf[p], gsems[p]
        )

    def wait_gather(p):
        pltpu.make_async_copy(
            w_hbm.at[idx_v.at[pl.ds(0, HBLK)]], rbuf[p], gsems[p]
        ).wait()

    def transpose_scale(p):
        @plsc.parallel_loop(0, HBLK, unroll=8)
        def _(r):
            lo = rbuf[p][r, pl.ds(0, LANES)] * SCALE
            hi = rbuf[p][r, pl.ds(LANES, LANES)] * SCALE
            plsc.store_scatter(tbuf[p], [iota128 + r], lo)
            plsc.store_scatter(tbuf[p], [iota128 + (LANES * HBLK + r)], hi)

    def start_store(k, p):
        u = wid * UNITS_B + k
        h = u // N_TC
        tc = u % N_TC
        for tr in range(N_TR):
            pltpu.async_copy(
                tbuf[p].at[pl.ds(tr * (8 * HBLK), 8 * HBLK)],
                out_hbm.at[h, tr, tc, :],
                ssems[p],
            )

    def wait_store(p):
        for tr in range(N_TR):
            pltpu.make_async_copy(
                tbuf[p].at[pl.ds(tr * (8 * HBLK), 8 * HBLK)],
                out_hbm.at[0, tr, 0, :],
                ssems[p],
            ).wait()

    start_gather(0, 0)

    def loop_body(t, carry):
        for p in range(2):
            k = t * 2 + p

            @pl.when(k + 1 < UNITS_B)
            def _():
                start_gather(k + 1, 1 - p)

            wait_gather(p)

            @pl.when(k >= 2)
            def _():
                wait_store(p)

            transpose_scale(p)
            start_store(k, p)
        return carry

    lax.fori_loop(0, UNITS_B // 2, loop_body, 0)

    for p in range(2):
        wait_store(p)


def kernel(x, W):
    wrow = _untile(jnp.transpose(W), W[TAIL_R0:].reshape(TAIL_N * EMB_DIM))
    out5 = _embed(wrow.reshape(VOC, EMB_DIM), jnp.transpose(x).reshape(TOTAL))
    out5 = out5.reshape(HIST, N_TR, N_TC, 8, HBLK)
    return jnp.transpose(out5, (2, 4, 0, 1, 3)).reshape(BATCH, HIST, EMB_DIM)


# speedup vs baseline: 1.6706x; 1.0073x over previous
import functools
import math

import jax
import jax.numpy as jnp
from jax import lax
from jax.experimental import pallas as pl
from jax.experimental.pallas import tpu as pltpu
from jax.experimental.pallas import tpu_sc as plsc

BATCH = 4096
HIST = 200
EMB_DIM = 32
VOC = 1000000
TOTAL = BATCH * HIST
SCALE = math.sqrt(EMB_DIM)

_info = plsc.get_sparse_core_info()
NC = _info.num_cores
NS = _info.num_subcores
LANES = _info.num_lanes
NW = NC * NS

_mesh = plsc.VectorSubcoreMesh(core_axis_name="c", subcore_axis_name="s")


RBLK = 512
N_FULL = (VOC // RBLK)
UNITS_EACH = N_FULL // NW
UNITS_REM = N_FULL % NW
TAIL_R0 = N_FULL * RBLK
TAIL_N = VOC - TAIL_R0


@functools.partial(
    pl.kernel,
    mesh=_mesh,
    compiler_params=pltpu.CompilerParams(
        use_tc_tiling_on_sc=True, needs_layout_passes=False
    ),
    out_type=jax.ShapeDtypeStruct((VOC * EMB_DIM,), jnp.float32),
    scratch_types=[
        [pltpu.VMEM((EMB_DIM, RBLK), jnp.float32) for _ in range(2)],
        [pltpu.VMEM((RBLK * EMB_DIM,), jnp.float32) for _ in range(2)],
        pltpu.VMEM((TAIL_N * EMB_DIM,), jnp.float32),
        [pltpu.SemaphoreType.DMA for _ in range(2)],
        [pltpu.SemaphoreType.DMA for _ in range(2)],
        pltpu.SemaphoreType.DMA,
    ],
)
def _untile(wt_hbm, wtail_hbm, wrow_hbm, tv, ov, tov, gsems, wsems, tsem):
    wid = lax.axis_index("s") * NC + lax.axis_index("c")
    iota = lax.iota(jnp.int32, LANES)
    iota_hi = iota + LANES

    def col_of(k):
        return k * NW + wid

    def start_read(k, p):
        tc = col_of(k)
        pltpu.async_copy(
            wt_hbm.at[:, pl.ds(tc * RBLK, RBLK)], tv[p], gsems[p]
        )

    def wait_read(p):
        pltpu.make_async_copy(
            wt_hbm.at[:, pl.ds(0, RBLK)], tv[p], gsems[p]
        ).wait()

    iota32 = iota * EMB_DIM

    def untile_block(p):
        @plsc.parallel_loop(0, (RBLK // LANES) * EMB_DIM, unroll=8)
        def _(j):
            d = j & (EMB_DIM - 1)
            blk = j >> 5
            vals = tv[p][d, pl.ds(blk * LANES, LANES)]
            plsc.store_scatter(
                ov[p], [iota32 + (blk * (LANES * EMB_DIM) + d)], vals
            )

    def start_write(k, p):
        tc = col_of(k)
        pltpu.async_copy(
            ov[p], wrow_hbm.at[pl.ds(tc * (RBLK * EMB_DIM), RBLK * EMB_DIM)],
            wsems[p],
        )

    def wait_write(p):
        pltpu.make_async_copy(
            ov[p], wrow_hbm.at[pl.ds(0, RBLK * EMB_DIM)], wsems[p]
        ).wait()

    n_units = UNITS_EACH + 1
    start_read(0, 0)

    def body(k, carry):
        for p in range(2):
            kk = k * 2 + p

            @pl.when(kk < UNITS_EACH)
            def _():
                @pl.when(kk + 1 < n_units)
                def _():
                    do_next = jnp.logical_or(
                        kk + 1 < UNITS_EACH, wid < UNITS_REM
                    )

                    @pl.when(do_next)
                    def _():
                        start_read(kk + 1, 1 - p)

                wait_read(p)

                @pl.when(kk >= 2)
                def _():
                    wait_write(p)

                untile_block(p)
                start_write(kk, p)
        return carry

    lax.fori_loop(0, (UNITS_EACH + 1) // 2, body, 0)

    p_x = UNITS_EACH % 2

    @pl.when(wid < UNITS_REM)
    def _():
        wait_read(p_x)
        wait_write(p_x)
        untile_block(p_x)
        start_write(UNITS_EACH, p_x)

    @pl.when(wid == NW - 1)
    def _():
        pltpu.sync_copy(wtail_hbm, tov)
        pltpu.async_copy(
            tov, wrow_hbm.at[pl.ds(TAIL_R0 * EMB_DIM, TAIL_N * EMB_DIM)],
            tsem,
        )
        pltpu.make_async_copy(
            tov, wrow_hbm.at[pl.ds(0, TAIL_N * EMB_DIM)], tsem
        ).wait()

    for p in range(2):
        wait_write(p)


PER_W = TOTAL // NW
HBLK = 128
N_TR = EMB_DIM // 8
N_TC = BATCH // HBLK
CHB = 512
TCPU = CHB // HBLK
UNITS_B = PER_W // CHB
UPH = BATCH // CHB


@functools.partial(
    pl.kernel,
    mesh=_mesh,
    compiler_params=pltpu.CompilerParams(
        use_tc_tiling_on_sc=False, needs_layout_passes=False
    ),
    out_type=jax.ShapeDtypeStruct(
        (HIST, N_TR, N_TC * 8 * HBLK), jnp.float32
    ),
    scratch_types=[
        pltpu.VMEM((PER_W,), jnp.int32),
        [pltpu.VMEM((CHB, EMB_DIM), jnp.float32) for _ in range(2)],
        [pltpu.VMEM((N_TR, TCPU * 8 * HBLK), jnp.float32) for _ in range(2)],
        [pltpu.SemaphoreType.DMA for _ in range(2)],
        [pltpu.SemaphoreType.DMA for _ in range(2)],
    ],
)
def _embed(w_hbm, x_hbm, out_hbm, idx_v, rbuf, tbuf, gsems, ssems):
    wid = lax.axis_index("s") * NC + lax.axis_index("c")
    base = wid * PER_W
    iota = lax.iota(jnp.int32, LANES)
    tr_lo = lax.shift_right_logical(iota, 3)
    tr_hi = tr_lo + 2
    in_c = (iota & 7) * HBLK

    pltpu.sync_copy(x_hbm.at[pl.ds(base, PER_W)], idx_v)

    def start_gather(k, p):
        pltpu.async_copy(
            w_hbm.at[idx_v.at[pl.ds(k * CHB, CHB)]], rbuf[p], gsems[p]
        )

    def wait_gather(p):
        pltpu.make_async_copy(
            w_hbm.at[idx_v.at[pl.ds(0, CHB)]], rbuf[p], gsems[p]
        ).wait()

    def transpose_scale(p):
        @plsc.parallel_loop(0, CHB, unroll=8)
        def _(j):
            s = (j >> 7) * (8 * HBLK) + (j & (HBLK - 1))
            lo = rbuf[p][j, pl.ds(0, LANES)] * SCALE
            hi = rbuf[p][j, pl.ds(LANES, LANES)] * SCALE
            plsc.store_scatter(tbuf[p], [tr_lo, in_c + s], lo)
            plsc.store_scatter(tbuf[p], [tr_hi, in_c + s], hi)

    def start_store(k, p):
        u = wid * UNITS_B + k
        h = u // UPH
        tc0 = (u % UPH) * TCPU
        pltpu.async_copy(
            tbuf[p],
            out_hbm.at[h, :, pl.ds(tc0 * (8 * HBLK), TCPU * 8 * HBLK)],
            ssems[p],
        )

    def wait_store(p):
        pltpu.make_async_copy(
            tbuf[p],
            out_hbm.at[0, :, pl.ds(0, TCPU * 8 * HBLK)],
            ssems[p],
        ).wait()

    start_gather(0, 0)

    def loop_body(t, carry):
        for p in range(2):
            k = t * 2 + p

            @pl.when(k + 1 < UNITS_B)
            def _():
                start_gather(k + 1, 1 - p)

            wait_gather(p)

            @pl.when(k >= 2)
            def _():
                wait_store(p)

            transpose_scale(p)
            start_store(k, p)
        return carry

    lax.fori_loop(0, UNITS_B // 2, loop_body, 0)

    for p in range(2):
        wait_store(p)


def kernel(x, W):
    wrow = _untile(jnp.transpose(W), W[TAIL_R0:].reshape(TAIL_N * EMB_DIM))
    out5 = _embed(wrow.reshape(VOC, EMB_DIM), jnp.transpose(x).reshape(TOTAL))
    out5 = out5.reshape(HIST, N_TR, N_TC, 8, HBLK)
    return jnp.transpose(out5, (2, 4, 0, 1, 3)).reshape(BATCH, HIST, EMB_DIM)


# speedup vs baseline: 2.4085x; 1.4418x over previous
import functools
import math

import jax
import jax.numpy as jnp
from jax import lax
from jax.experimental import pallas as pl
from jax.experimental.pallas import tpu as pltpu
from jax.experimental.pallas import tpu_sc as plsc

BATCH = 4096
HIST = 200
EMB_DIM = 32
VOC = 1000000
TOTAL = BATCH * HIST
SCALE = math.sqrt(EMB_DIM)

_info = plsc.get_sparse_core_info()
NC = _info.num_cores
NS = _info.num_subcores
LANES = _info.num_lanes
NW = NC * NS

_mesh = plsc.VectorSubcoreMesh(core_axis_name="c", subcore_axis_name="s")


RBLK = 512
N_FULL = (VOC // RBLK)
UNITS_EACH = N_FULL // NW
UNITS_REM = N_FULL % NW
TAIL_R0 = N_FULL * RBLK
TAIL_N = VOC - TAIL_R0


@functools.partial(
    pl.kernel,
    mesh=_mesh,
    compiler_params=pltpu.CompilerParams(
        use_tc_tiling_on_sc=True, needs_layout_passes=False
    ),
    out_type=jax.ShapeDtypeStruct((VOC * EMB_DIM,), jnp.float32),
    scratch_types=[
        [pltpu.VMEM((EMB_DIM, RBLK + 1), jnp.float32) for _ in range(2)],
        [pltpu.VMEM((RBLK * EMB_DIM,), jnp.float32) for _ in range(2)],
        pltpu.VMEM((TAIL_N * EMB_DIM,), jnp.float32),
        [pltpu.SemaphoreType.DMA for _ in range(2)],
        [pltpu.SemaphoreType.DMA for _ in range(2)],
        pltpu.SemaphoreType.DMA,
    ],
)
def _untile(wt_hbm, wtail_hbm, wrow_hbm, tv, ov, tov, gsems, wsems, tsem):
    wid = lax.axis_index("s") * NC + lax.axis_index("c")
    iota = lax.iota(jnp.int32, LANES)
    iota_hi = iota + LANES

    def col_of(k):
        return k * NW + wid

    def start_read(k, p):
        tc = col_of(k)
        pltpu.async_copy(
            wt_hbm.at[:, pl.ds(tc * RBLK, RBLK)],
            tv[p].at[:, pl.ds(0, RBLK)],
            gsems[p],
        )

    def wait_read(p):
        pltpu.make_async_copy(
            wt_hbm.at[:, pl.ds(0, RBLK)],
            tv[p].at[:, pl.ds(0, RBLK)],
            gsems[p],
        ).wait()

    def untile_block(p):
        @plsc.parallel_loop(0, RBLK, unroll=8)
        def _(rs):
            cs = jnp.full((LANES,), rs, jnp.int32)
            lo = plsc.load_gather(tv[p], [iota, cs])
            hi = plsc.load_gather(tv[p], [iota_hi, cs])
            ov[p][pl.ds(rs * EMB_DIM, LANES)] = lo
            ov[p][pl.ds(rs * EMB_DIM + LANES, LANES)] = hi

    def start_write(k, p):
        tc = col_of(k)
        pltpu.async_copy(
            ov[p], wrow_hbm.at[pl.ds(tc * (RBLK * EMB_DIM), RBLK * EMB_DIM)],
            wsems[p],
        )

    def wait_write(p):
        pltpu.make_async_copy(
            ov[p], wrow_hbm.at[pl.ds(0, RBLK * EMB_DIM)], wsems[p]
        ).wait()

    n_units = UNITS_EACH + 1
    start_read(0, 0)

    def body(k, carry):
        for p in range(2):
            kk = k * 2 + p

            @pl.when(kk < UNITS_EACH)
            def _():
                @pl.when(kk + 1 < n_units)
                def _():
                    do_next = jnp.logical_or(
                        kk + 1 < UNITS_EACH, wid < UNITS_REM
                    )

                    @pl.when(do_next)
                    def _():
                        start_read(kk + 1, 1 - p)

                wait_read(p)

                @pl.when(kk >= 2)
                def _():
                    wait_write(p)

                untile_block(p)
                start_write(kk, p)
        return carry

    lax.fori_loop(0, (UNITS_EACH + 1) // 2, body, 0)

    p_x = UNITS_EACH % 2

    @pl.when(wid < UNITS_REM)
    def _():
        wait_read(p_x)
        wait_write(p_x)
        untile_block(p_x)
        start_write(UNITS_EACH, p_x)

    @pl.when(wid == NW - 1)
    def _():
        pltpu.sync_copy(wtail_hbm, tov)
        pltpu.async_copy(
            tov, wrow_hbm.at[pl.ds(TAIL_R0 * EMB_DIM, TAIL_N * EMB_DIM)],
            tsem,
        )
        pltpu.make_async_copy(
            tov, wrow_hbm.at[pl.ds(0, TAIL_N * EMB_DIM)], tsem
        ).wait()

    for p in range(2):
        wait_write(p)


PER_W = TOTAL // NW
HBLK = 128
N_TR = EMB_DIM // 8
N_TC = BATCH // HBLK
CHB = 512
TCPU = CHB // HBLK
UNITS_B = PER_W // CHB
UPH = BATCH // CHB


@functools.partial(
    pl.kernel,
    mesh=_mesh,
    compiler_params=pltpu.CompilerParams(
        use_tc_tiling_on_sc=False, needs_layout_passes=False
    ),
    out_type=jax.ShapeDtypeStruct(
        (HIST, N_TR, N_TC, 8, HBLK), jnp.float32
    ),
    scratch_types=[
        pltpu.VMEM((PER_W,), jnp.int32),
        [pltpu.VMEM((CHB, EMB_DIM), jnp.float32) for _ in range(2)],
        [pltpu.VMEM((N_TR * TCPU * 8, HBLK + 1), jnp.float32) for _ in range(2)],
        [pltpu.SemaphoreType.DMA for _ in range(2)],
        [pltpu.SemaphoreType.DMA for _ in range(2)],
    ],
)
def _embed(w_hbm, x_hbm, out_hbm, idx_v, rbuf, tbuf, gsems, ssems):
    wid = lax.axis_index("s") * NC + lax.axis_index("c")
    base = wid * PER_W
    iota = lax.iota(jnp.int32, LANES)
    row_lo = lax.shift_right_logical(iota, 3) * (TCPU * 8) + (iota & 7)
    row_hi = row_lo + 2 * (TCPU * 8)

    pltpu.sync_copy(x_hbm.at[pl.ds(base, PER_W)], idx_v)

    def start_gather(k, p):
        pltpu.async_copy(
            w_hbm.at[idx_v.at[pl.ds(k * CHB, CHB)]], rbuf[p], gsems[p]
        )

    def wait_gather(p):
        pltpu.make_async_copy(
            w_hbm.at[idx_v.at[pl.ds(0, CHB)]], rbuf[p], gsems[p]
        ).wait()

    def transpose_scale(p):
        @plsc.parallel_loop(0, CHB, unroll=8)
        def _(j):
            tcl8 = lax.shift_right_logical(j, 7) * 8
            cs = jnp.full((LANES,), j & (HBLK - 1), jnp.int32)
            lo = rbuf[p][j, pl.ds(0, LANES)] * SCALE
            hi = rbuf[p][j, pl.ds(LANES, LANES)] * SCALE
            plsc.store_scatter(tbuf[p], [row_lo + tcl8, cs], lo)
            plsc.store_scatter(tbuf[p], [row_hi + tcl8, cs], hi)

    def start_store(k, p):
        u = wid * UNITS_B + k
        h = u // UPH
        tc0 = (u % UPH) * TCPU
        for tr in range(N_TR):
            for tcl in range(TCPU):
                pltpu.async_copy(
                    tbuf[p].at[
                        pl.ds(tr * (TCPU * 8) + tcl * 8, 8), pl.ds(0, HBLK)
                    ],
                    out_hbm.at[h, tr, tc0 + tcl, :, :],
                    ssems[p],
                )

    def wait_store(p):
        for _ in range(N_TR * TCPU):
            pltpu.make_async_copy(
                tbuf[p].at[pl.ds(0, 8), pl.ds(0, HBLK)],
                out_hbm.at[0, 0, 0, :, :],
                ssems[p],
            ).wait()

    start_gather(0, 0)

    def loop_body(t, carry):
        for p in range(2):
            k = t * 2 + p

            @pl.when(k + 1 < UNITS_B)
            def _():
                start_gather(k + 1, 1 - p)

            wait_gather(p)

            @pl.when(k >= 2)
            def _():
                wait_store(p)

            transpose_scale(p)
            start_store(k, p)
        return carry

    lax.fori_loop(0, UNITS_B // 2, loop_body, 0)

    for p in range(2):
        wait_store(p)


def kernel(x, W):
    wrow = _untile(jnp.transpose(W), W[TAIL_R0:].reshape(TAIL_N * EMB_DIM))
    out5 = _embed(wrow.reshape(VOC, EMB_DIM), jnp.transpose(x).reshape(TOTAL))
    return jnp.transpose(out5, (2, 4, 0, 1, 3)).reshape(BATCH, HIST, EMB_DIM)
